# baseline (speedup 1.0000x reference)
import functools

import jax
import jax.numpy as jnp
from jax import lax
from jax.experimental import pallas as pl
from jax.experimental.pallas import tpu as pltpu
from jax.experimental.pallas import tpu_sc as plsc

_NC = 2
_NS = 16
_LN = 16
_NW = _NC * _NS

_UNROLL = 4


def _fold_pack_body(emb_ref, lin_ref, q_ref, s_ref):
    m = lax.dot_general(
        emb_ref[...], lin_ref[...], (((1,), (1,)), ((), ())),
        preferred_element_type=jnp.float32)
    nclass = m.shape[1]
    nword = (nclass + 3) // 4
    a = jnp.max(jnp.max(jnp.abs(m), axis=0, keepdims=True),
                axis=1, keepdims=True)
    a = jnp.maximum(a, 1e-30)
    scale = a * (1.0 / 127.0)
    q = jnp.clip(jnp.round(m * (1.0 / scale)) + 128.0, 0.0, 255.0)
    w = None
    dn = (((1,), (0,)), ((), ()))
    for b in range(4):
        ci = lax.broadcasted_iota(jnp.int32, (nclass, nword), 0)
        ck = lax.broadcasted_iota(jnp.int32, (nclass, nword), 1)
        sel = (ci == 4 * ck + b).astype(jnp.float32)
        wb = lax.dot_general(q, sel, dn,
                             preferred_element_type=jnp.float32)
        wb = wb.astype(jnp.int32) << (8 * b)
        w = wb if w is None else (w | wb)
    q_ref[...] = w
    s_ref[...] = scale


def _fold_pack_table(emb, lin):
    nword = (lin.shape[0] + 3) // 4
    return pl.pallas_call(
        _fold_pack_body,
        out_shape=[
            jax.ShapeDtypeStruct((emb.shape[0], nword), jnp.int32),
            jax.ShapeDtypeStruct((1, 1), jnp.float32),
        ],
    )(emb, lin)


_SG = 128
_REP = 16


def _make_sc_bow(batch, seq_len, vocab, nclass):
    nword = (nclass + 3) // 4
    docs_per_w = batch // _NW
    sgroups = docs_per_w // _SG
    subs = _SG // _LN
    stride = nword * _REP

    @functools.partial(
        pl.kernel,
        out_type=jax.ShapeDtypeStruct((nclass * batch,), jnp.float32),
        mesh=plsc.VectorSubcoreMesh(core_axis_name="c", subcore_axis_name="s"),
        compiler_params=pltpu.CompilerParams(needs_layout_passes=False),
        scratch_types=[
            pltpu.VMEM((vocab * stride,), jnp.int32),
            pltpu.VMEM((_LN,), jnp.float32),
            pltpu.VMEM((seq_len, _SG), jnp.int32),
            pltpu.VMEM((seq_len, _SG), jnp.int32),
            pltpu.VMEM((nclass * docs_per_w,), jnp.float32),
            pltpu.SemaphoreType.DMA,
            pltpu.SemaphoreType.DMA,
        ],
    )
    def sc_bow(docs_hbm, m_hbm, scale_hbm, out_hbm, m_v, scale_v,
               docs_v0, docs_v1, out_v, sem0, sem1):
        wid = lax.axis_index("s") * _NC + lax.axis_index("c")
        base = wid * docs_per_w
        pltpu.sync_copy(m_hbm, m_v)
        pltpu.sync_copy(scale_hbm, scale_v)
        sv = scale_v[...]
        iota = lax.iota(jnp.int32, _LN)
        k_off = [jnp.int32(k * _REP) + iota for k in range(nword)]
        lomask = jnp.int32(0x00FF00FF)
        debias = jnp.float32(128.0 * seq_len)

        def docs_dma(sg, buf_ref, sem):
            return pltpu.make_async_copy(
                docs_hbm.at[:, pl.ds(base + sg * _SG, _SG)], buf_ref, sem)

        def accumulate(sg, buf_ref):
            for sub in range(subs):
                col0 = sub * _LN

                def l_body(i, accs):
                    new = list(accs)
                    for j in range(_UNROLL):
                        l = i * _UNROLL + j
                        toks = buf_ref[l, pl.ds(col0, _LN)]
                        tb = toks * stride
                        for k in range(nword):
                            w = plsc.load_gather(m_v, [tb + k_off[k]])
                            new[2 * k] = new[2 * k] + (w & lomask)
                            new[2 * k + 1] = new[2 * k + 1] + (
                                lax.shift_right_logical(w, 8) & lomask)
                    return tuple(new)

                accs = lax.fori_loop(
                    0, seq_len // _UNROLL, l_body,
                    tuple(jnp.zeros((_LN,), jnp.int32)
                          for _ in range(2 * nword)))
                lrow = sg * _SG + sub * _LN + iota
                for c in range(nclass):
                    k, b = c // 4, c % 4
                    acc = accs[2 * k + (b & 1)]
                    cnt = (lax.shift_right_logical(acc, 16) if b >= 2
                           else (acc & jnp.int32(0xFFFF)))
                    f = (cnt.astype(jnp.float32) - debias) * sv
                    plsc.store_scatter(out_v, [c * docs_per_w + lrow], f)

        docs_dma(0, docs_v0, sem0).start()

        def g2_body(h, carry):
            g0 = h * 2

            @pl.when(g0 + 1 < sgroups)
            def _():
                docs_dma(g0 + 1, docs_v1, sem1).start()

            docs_dma(g0, docs_v0, sem0).wait()
            accumulate(g0, docs_v0)

            @pl.when(g0 + 2 < sgroups)
            def _():
                docs_dma(g0 + 2, docs_v0, sem0).start()

            @pl.when(g0 + 1 < sgroups)
            def _():
                docs_dma(g0 + 1, docs_v1, sem1).wait()
                accumulate(g0 + 1, docs_v1)

            return carry

        lax.fori_loop(0, (sgroups + 1) // 2, g2_body, 0)
        for c in range(nclass):
            pltpu.sync_copy(
                out_v.at[pl.ds(c * docs_per_w, docs_per_w)],
                out_hbm.at[pl.ds(c * batch + base, docs_per_w)])

    return sc_bow


def _lsm_body(inv_len, s_ref, b_ref, o_ref):
    z = s_ref[...] * inv_len + b_ref[...]
    m = jnp.max(z, axis=1, keepdims=True)
    e = jnp.exp(z - m)
    lse = jnp.log(jnp.sum(e, axis=1, keepdims=True))
    o_ref[...] = z - m - lse


def kernel(docs, emb_weight, lin_weight, lin_bias):
    batch, seq_len = docs.shape
    vocab, _ = emb_weight.shape
    nclass = lin_weight.shape[0]

    q, scale = _fold_pack_table(emb_weight, lin_weight)
    nword = (nclass + 3) // 4
    q_rep = jnp.broadcast_to(
        q.reshape(-1)[:, None], (vocab * nword, _REP)).reshape(-1)
    scale_vec = jnp.broadcast_to(scale.reshape(1), (_LN,))
    docs_t = docs.T
    sc_bow = _make_sc_bow(batch, seq_len, vocab, nclass)
    s_t = sc_bow(docs_t, q_rep, scale_vec).reshape(nclass, batch)
    out_t = pl.pallas_call(
        functools.partial(_lsm_body, 1.0 / seq_len),
        out_shape=jax.ShapeDtypeStruct((nclass, batch), jnp.float32),
    )(s_t, lin_bias.reshape(nclass, 1))
    return out_t.T


# speedup vs baseline: 1.0676x; 1.0676x over previous
import functools

import jax
import jax.numpy as jnp
from jax import lax
from jax.experimental import pallas as pl
from jax.experimental.pallas import tpu as pltpu
from jax.experimental.pallas import tpu_sc as plsc

_NC = 2
_NS = 16
_LN = 16
_NW = _NC * _NS

_UNROLL = 4


def _fold_pack_body(rep, emb_ref, lin_ref, m_ref):
    m = lax.dot_general(
        emb_ref[...], lin_ref[...], (((1,), (1,)), ((), ())),
        preferred_element_type=jnp.float32)
    nclass = m.shape[1]
    npair = nclass // 2
    ci = lax.broadcasted_iota(jnp.int32, (nclass, npair * rep), 0)
    cj = lax.broadcasted_iota(jnp.int32, (nclass, npair * rep), 1)
    sel_even = (ci == 2 * (cj // rep)).astype(jnp.float32)
    sel_odd = (ci == 2 * (cj // rep) + 1).astype(jnp.float32)
    dn = (((1,), (0,)), ((), ()))
    lo = lax.dot_general(m, sel_even, dn, preferred_element_type=jnp.float32)
    hi = lax.dot_general(m, sel_odd, dn, preferred_element_type=jnp.float32)
    ulo = lax.bitcast_convert_type(
        lo.astype(jnp.bfloat16), jnp.uint16).astype(jnp.uint32)
    uhi = lax.bitcast_convert_type(
        hi.astype(jnp.bfloat16), jnp.uint16).astype(jnp.uint32)
    m_ref[...] = (ulo | (uhi << 16)).astype(jnp.int32)


def _fold_pack_table(emb, lin, rep):
    return pl.pallas_call(
        functools.partial(_fold_pack_body, rep),
        out_shape=jax.ShapeDtypeStruct(
            (emb.shape[0], lin.shape[0] // 2 * rep), jnp.int32),
    )(emb, lin)


_SG = 128
_REP = 8


def _make_sc_bow(batch, seq_len, vocab, nclass):
    npair = nclass // 2
    docs_per_w = batch // _NW
    sgroups = docs_per_w // _SG
    subs = _SG // _LN

    @functools.partial(
        pl.kernel,
        out_type=jax.ShapeDtypeStruct((nclass * batch,), jnp.float32),
        mesh=plsc.VectorSubcoreMesh(core_axis_name="c", subcore_axis_name="s"),
        compiler_params=pltpu.CompilerParams(needs_layout_passes=False),
        scratch_types=[
            pltpu.VMEM((vocab * npair * _REP,), jnp.int32),
            pltpu.VMEM((seq_len, _SG), jnp.int32),
            pltpu.VMEM((seq_len, _SG), jnp.int32),
            pltpu.VMEM((nclass * docs_per_w,), jnp.float32),
            pltpu.SemaphoreType.DMA,
            pltpu.SemaphoreType.DMA,
        ],
    )
    def sc_bow(docs_hbm, m_hbm, out_hbm, m_v, docs_v0, docs_v1, out_v,
               sem0, sem1):
        wid = lax.axis_index("s") * _NC + lax.axis_index("c")
        base = wid * docs_per_w
        pltpu.sync_copy(m_hbm, m_v)
        iota = lax.iota(jnp.int32, _LN)
        lane_res = iota & jnp.int32(_REP - 1)
        k_off = [jnp.int32(k * _REP) + lane_res for k in range(npair)]

        def docs_dma(sg, buf_ref, sem):
            return pltpu.make_async_copy(
                docs_hbm.at[:, pl.ds(base + sg * _SG, _SG)], buf_ref, sem)

        def accumulate(sg, buf_ref):
            for sub in range(subs):
                col0 = sub * _LN

                def l_body(i, accs):
                    new = list(accs)
                    for j in range(_UNROLL):
                        l = i * _UNROLL + j
                        toks = buf_ref[l, pl.ds(col0, _LN)]
                        tb = toks * (npair * _REP)
                        for k in range(npair):
                            w = plsc.load_gather(m_v, [tb + k_off[k]])
                            new[k] = new[k] + plsc.bitcast(w, jnp.bfloat16)
                    return tuple(new)

                accs = lax.fori_loop(
                    0, seq_len // _UNROLL, l_body,
                    tuple(jnp.zeros((2 * _LN,), jnp.bfloat16)
                          for _ in range(npair)))
                lrow = sg * _SG + sub * _LN + iota
                for k in range(npair):
                    w = plsc.bitcast(accs[k], jnp.int32)
                    f_even = plsc.bitcast(w << 16, jnp.float32)
                    f_odd = plsc.bitcast(w & jnp.int32(-65536), jnp.float32)
                    plsc.store_scatter(
                        out_v, [(2 * k) * docs_per_w + lrow], f_even)
                    plsc.store_scatter(
                        out_v, [(2 * k + 1) * docs_per_w + lrow], f_odd)

        docs_dma(0, docs_v0, sem0).start()

        def g2_body(h, carry):
            g0 = h * 2

            @pl.when(g0 + 1 < sgroups)
            def _():
                docs_dma(g0 + 1, docs_v1, sem1).start()

            docs_dma(g0, docs_v0, sem0).wait()
            accumulate(g0, docs_v0)

            @pl.when(g0 + 2 < sgroups)
            def _():
                docs_dma(g0 + 2, docs_v0, sem0).start()

            @pl.when(g0 + 1 < sgroups)
            def _():
                docs_dma(g0 + 1, docs_v1, sem1).wait()
                accumulate(g0 + 1, docs_v1)

            return carry

        lax.fori_loop(0, (sgroups + 1) // 2, g2_body, 0)
        for c in range(nclass):
            pltpu.sync_copy(
                out_v.at[pl.ds(c * docs_per_w, docs_per_w)],
                out_hbm.at[pl.ds(c * batch + base, docs_per_w)])

    return sc_bow


def _lsm_body(inv_len, s_ref, b_ref, o_ref):
    z = s_ref[...] * inv_len + b_ref[...]
    m = jnp.max(z, axis=1, keepdims=True)
    e = jnp.exp(z - m)
    lse = jnp.log(jnp.sum(e, axis=1, keepdims=True))
    o_ref[...] = z - m - lse


def kernel(docs, emb_weight, lin_weight, lin_bias):
    batch, seq_len = docs.shape
    vocab, _ = emb_weight.shape
    nclass = lin_weight.shape[0]

    m_rep = _fold_pack_table(emb_weight, lin_weight, _REP).reshape(-1)
    docs_t = docs.T
    sc_bow = _make_sc_bow(batch, seq_len, vocab, nclass)
    s_t = sc_bow(docs_t, m_rep).reshape(nclass, batch)
    out_t = pl.pallas_call(
        functools.partial(_lsm_body, 1.0 / seq_len),
        out_shape=jax.ShapeDtypeStruct((nclass, batch), jnp.float32),
    )(s_t, lin_bias.reshape(nclass, 1))
    return out_t.T
